# baseline (speedup 1.0000x reference)
import functools

import jax
import jax.numpy as jnp
from jax import lax
from jax.experimental import pallas as pl
from jax.experimental.pallas import tpu as pltpu
from jax.experimental.pallas import tpu_sc as plsc

EMBED = 64
_B = 1024
_L = 200

_info = plsc.get_sparse_core_info()
_NC, _NS = _info.num_cores, _info.num_subcores
_NW = _NC * _NS
_STREAM = 128
_SUB = 5
_CHUNK = _STREAM * _SUB
_NT = _B * _L
_B_PER_W = _NT // _NW
_NSTR = _B_PER_W // _STREAM
_NCHUNK = _B_PER_W // _CHUNK

_mesh = plsc.VectorSubcoreMesh(core_axis_name="c", subcore_axis_name="s")


@functools.partial(
    pl.kernel,
    mesh=_mesh,
    out_type=jax.ShapeDtypeStruct((_NT, EMBED), jnp.float32),
    compiler_params=pltpu.CompilerParams(use_tc_tiling_on_sc=False),
    scratch_types=[
        pltpu.VMEM((_NSTR, _STREAM), jnp.int32),
        pltpu.VMEM((_CHUNK, EMBED), jnp.float32),
        pltpu.VMEM((_CHUNK, EMBED), jnp.float32),
        pltpu.SemaphoreType.DMA,
        pltpu.SemaphoreType.DMA,
        pltpu.SemaphoreType.DMA,
        pltpu.SemaphoreType.DMA,
    ],
)
def _gather(idx_hbm, table_hbm, out_hbm, idx_v, rows0, rows1,
            gs0, gs1, os0, os1):
    wid = lax.axis_index("s") * _NC + lax.axis_index("c")
    base = wid * _B_PER_W
    bufs = ((rows0, gs0, os0), (rows1, gs1, os1))
    pltpu.sync_copy(idx_hbm.at[wid], idx_v)

    def fire_gathers(c, b):
        rows, gs, _ = bufs[b]
        for s in range(_SUB):
            pltpu.async_copy(table_hbm.at[idx_v.at[c * _SUB + s]],
                             rows.at[pl.ds(s * _STREAM, _STREAM)], gs)

    def drain_gathers(b):
        rows, gs, _ = bufs[b]
        pltpu.make_async_copy(table_hbm.at[pl.ds(0, _CHUNK)], rows, gs).wait()

    def out_copy(c, b):
        rows, _, os = bufs[b]
        return pltpu.async_copy(
            rows, out_hbm.at[pl.ds(base + c * _CHUNK, _CHUNK)], os)

    fire_gathers(0, 0)
    fire_gathers(1, 1)

    def body(j, carry):
        c0 = 2 * j
        c1 = c0 + 1
        drain_gathers(0)
        o0 = out_copy(c0, 0)
        drain_gathers(1)
        o1 = out_copy(c1, 1)
        o0.wait()
        fire_gathers(c0 + 2, 0)
        o1.wait()
        fire_gathers(c1 + 2, 1)
        return carry

    lax.fori_loop(0, _NCHUNK // 2 - 1, body, 0)

    drain_gathers(0)
    o0 = out_copy(_NCHUNK - 2, 0)
    drain_gathers(1)
    o1 = out_copy(_NCHUNK - 1, 1)
    o0.wait()
    o1.wait()


def kernel(inputtokens, table):
    idx = inputtokens.reshape(_NW, _NSTR, _STREAM).astype(jnp.int32)
    table_lin = jnp.concatenate(
        [table[0::2], table[1::2]], axis=1).reshape(1000000, EMBED)
    out = _gather(idx, table_lin)
    return out.reshape(_B, _L, EMBED)


# speedup vs baseline: 9.3174x; 9.3174x over previous
import functools

import jax
import jax.numpy as jnp
from jax import lax
from jax.experimental import pallas as pl
from jax.experimental.pallas import tpu as pltpu
from jax.experimental.pallas import tpu_sc as plsc

EMBED = 64
_B = 1024
_L = 200

_info = plsc.get_sparse_core_info()
_NC, _NS = _info.num_cores, _info.num_subcores
_NW = _NC * _NS
_BLK = 128
_JB = _B // _BLK
_NBLK = _L * _JB
_BPW = _NBLK // _NW
_NS_ = 5

_mesh = plsc.VectorSubcoreMesh(core_axis_name="c", subcore_axis_name="s")


@functools.partial(
    pl.kernel,
    mesh=_mesh,
    out_type=jax.ShapeDtypeStruct((_L, 8, _JB, 8, _BLK), jnp.float32),
    compiler_params=pltpu.CompilerParams(
        use_tc_tiling_on_sc=False, needs_layout_passes=False),
    scratch_types=[
        pltpu.VMEM((_NS_, _BLK), jnp.int32),
        pltpu.VMEM((_NS_, _BLK), jnp.int32),
        pltpu.VMEM((_NS_, _BLK, 128), jnp.float32),
        pltpu.VMEM((_NS_, 8, 8, _BLK), jnp.float32),
        pltpu.SemaphoreType.DMA((_NS_,)),
        pltpu.SemaphoreType.DMA((_NS_,)),
        pltpu.SemaphoreType.DMA((_NS_,)),
    ],
)
def _gather(idx_hbm, table2_hbm, out_hbm, idx_vv, pidx_vv, pair_vv, out_vv,
            isem, gsem, osem):
    wid = lax.axis_index("s") * _NC + lax.axis_index("c")

    def lj(i):
        blk = wid * _BPW + i
        return blk // _JB, blk % _JB

    def fire_front(i, p):
        l, jb = lj(i)
        idx_p = idx_vv.at[p]
        pidx_p = pidx_vv.at[p]
        pltpu.async_copy(idx_hbm.at[l, pl.ds(jb * _BLK, _BLK)], idx_p,
                         isem.at[p]).wait()
        for g in range(_BLK // 16):
            pidx_p[pl.ds(g * 16, 16)] = idx_p[pl.ds(g * 16, 16)] >> 1
        pltpu.async_copy(table2_hbm.at[pidx_p], pair_vv.at[p], gsem.at[p])

    def drain_gather(p):
        pltpu.make_async_copy(table2_hbm.at[pl.ds(0, _BLK)], pair_vv.at[p],
                              gsem.at[p]).wait()

    def drain_out(p):
        pltpu.make_async_copy(out_vv.at[p], out_hbm.at[0, :, 0, :, :],
                              osem.at[p]).wait()

    def back(i, p):
        l, jb = lj(i)
        idx_p = idx_vv.at[p]
        pair_p = pair_vv.at[p]
        out_p = out_vv.at[p]
        drain_gather(p)
        h64s = [(idx_p[pl.ds(g * 16, 16)] & 1) * 64
                for g in range(_BLK // 16)]
        b_ids = [lax.iota(jnp.int32, 16) + g * 16 for g in range(_BLK // 16)]

        def er_body(er, carry):
            for es in range(8):
                e = er * 8 + es
                for g in range(_BLK // 16):
                    out_p[er, es, pl.ds(g * 16, 16)] = plsc.load_gather(
                        pair_p, [b_ids[g], h64s[g] + e])
            return carry

        lax.fori_loop(0, 8, er_body, 0)
        pltpu.async_copy(out_p, out_hbm.at[l, :, jb, :, :], osem.at[p])

    for k in range(_NS_):
        l, jb = lj(k)
        pltpu.async_copy(out_vv.at[k], out_hbm.at[l, :, jb, :, :], osem.at[k])
    for k in range(4):
        fire_front(k, k)

    def body(j, carry):
        i0 = _NS_ * j
        for k in range(_NS_):
            drain_out(k)
            back(i0 + k, k)
            fire_front(jnp.minimum(i0 + k + 4, _BPW - 1), (k + 4) % _NS_)
        return carry

    lax.fori_loop(0, _BPW // _NS_, body, 0)

    for k in range(4):
        drain_gather(k)
    for k in range(_NS_):
        drain_out(k)


def kernel(inputtokens, table):
    idxT = jnp.transpose(inputtokens).astype(jnp.int32)
    table2 = table.reshape(500000, 128)
    out5 = _gather(idxT, table2)
    out = jnp.transpose(out5, (2, 4, 0, 1, 3))
    return out.reshape(_B, _L, EMBED)


# speedup vs baseline: 11.5714x; 1.2419x over previous
import functools

import jax
import jax.numpy as jnp
from jax import lax
from jax.experimental import pallas as pl
from jax.experimental.pallas import tpu as pltpu
from jax.experimental.pallas import tpu_sc as plsc

EMBED = 64
_B = 1024
_L = 200

_info = plsc.get_sparse_core_info()
_NC, _NS = _info.num_cores, _info.num_subcores
_NW = _NC * _NS
_STREAM = 128
_SUB = 5
_CHUNK = _STREAM * _SUB
_NT = _B * _L
_B_PER_W = _NT // _NW
_NSTR = _B_PER_W // _STREAM
_NCHUNK = _B_PER_W // _CHUNK

_mesh = plsc.VectorSubcoreMesh(core_axis_name="c", subcore_axis_name="s")


@functools.partial(
    pl.kernel,
    mesh=_mesh,
    out_type=jax.ShapeDtypeStruct((_NT, EMBED), jnp.float32),
    compiler_params=pltpu.CompilerParams(use_tc_tiling_on_sc=False),
    scratch_types=[
        pltpu.VMEM((_NSTR, _STREAM), jnp.int32),
        pltpu.VMEM((_CHUNK, EMBED), jnp.float32),
        pltpu.VMEM((_CHUNK, EMBED), jnp.float32),
        pltpu.SemaphoreType.DMA,
        pltpu.SemaphoreType.DMA,
        pltpu.SemaphoreType.DMA,
        pltpu.SemaphoreType.DMA,
    ],
)
def _gather(idx_hbm, table_hbm, out_hbm, idx_v, rows0, rows1,
            gs0, gs1, os0, os1):
    wid = lax.axis_index("s") * _NC + lax.axis_index("c")
    base = wid * _B_PER_W
    bufs = ((rows0, gs0, os0), (rows1, gs1, os1))
    pltpu.sync_copy(idx_hbm.at[wid], idx_v)

    def fire_gathers(c, b):
        rows, gs, _ = bufs[b]
        for s in range(_SUB):
            pltpu.async_copy(table_hbm.at[idx_v.at[c * _SUB + s]],
                             rows.at[pl.ds(s * _STREAM, _STREAM)], gs)

    def drain_gathers(b):
        rows, gs, _ = bufs[b]
        pltpu.make_async_copy(table_hbm.at[pl.ds(0, _CHUNK)], rows, gs).wait()

    def out_copy(c, b):
        rows, _, os = bufs[b]
        return pltpu.async_copy(
            rows, out_hbm.at[pl.ds(base + c * _CHUNK, _CHUNK)], os)

    fire_gathers(0, 0)
    fire_gathers(1, 1)

    def body(j, carry):
        c0 = 2 * j
        c1 = c0 + 1
        drain_gathers(0)
        o0 = out_copy(c0, 0)
        drain_gathers(1)
        o1 = out_copy(c1, 1)
        o0.wait()
        fire_gathers(c0 + 2, 0)
        o1.wait()
        fire_gathers(c1 + 2, 1)
        return carry

    lax.fori_loop(0, _NCHUNK // 2 - 1, body, 0)

    drain_gathers(0)
    o0 = out_copy(_NCHUNK - 2, 0)
    drain_gathers(1)
    o1 = out_copy(_NCHUNK - 1, 1)
    o0.wait()
    o1.wait()


def kernel(inputtokens, table):
    idx = inputtokens.reshape(_NW, _NSTR, _STREAM).astype(jnp.int32)
    out = _gather(idx, table)
    return out.reshape(_B, _L, EMBED)
